# baseline (speedup 1.0000x reference)
import functools

import jax
import jax.numpy as jnp
from jax import lax
from jax.experimental import pallas as pl
from jax.experimental.pallas import tpu as pltpu
from jax.experimental.pallas import tpu_sc as plsc

_D = 64
_NC = 2
_NS = 16
_NW = _NC * _NS
_L = 16


def _sc_gather(pairs, idsf, b, k):
    n = b * k
    bb = b // _NW
    nbt = bb // 128
    nchunk = k * nbt
    bpw = n // _NW
    mesh = plsc.VectorSubcoreMesh(core_axis_name="c", subcore_axis_name="s")

    @functools.partial(
        pl.kernel,
        out_type=jax.ShapeDtypeStruct((k, _D // 8, b // 128, 8, 128), jnp.float32),
        mesh=mesh,
        scratch_types=[
            pltpu.VMEM((bpw,), jnp.int32),
            pltpu.VMEM((bpw,), jnp.int32),
            pltpu.VMEM((128, 128), jnp.float32),
            pltpu.VMEM((128, 128), jnp.float32),
            pltpu.VMEM((8, 1, 8, 128), jnp.float32),
            pltpu.VMEM((8, 1, 8, 128), jnp.float32),
            pltpu.SemaphoreType.DMA,
            pltpu.SemaphoreType.DMA,
            pltpu.SemaphoreType.DMA,
            pltpu.SemaphoreType.DMA,
        ],
        compiler_params=pltpu.CompilerParams(
            use_tc_tiling_on_sc=False, needs_layout_passes=False
        ),
    )
    def body(pairs_hbm, ids_hbm, out_hbm, pidx_v, poff_v, rows0, rows1,
             tr0, tr1, gsem0, gsem1, osem0, osem1):
        wid = lax.axis_index("s") * _NC + lax.axis_index("c")
        bbase = wid * bb

        for k_ in range(k):
            pltpu.sync_copy(
                ids_hbm.at[pl.ds(k_ * b + bbase, bb)],
                pidx_v.at[pl.ds(k_ * bb, bb)],
            )

        @pl.loop(0, bpw // _L)
        def split(i):
            v = pidx_v[pl.ds(i * _L, _L)]
            poff_v[pl.ds(i * _L, _L)] = (v & 1) * _D
            pidx_v[pl.ds(i * _L, _L)] = lax.shift_right_logical(v, 1)

        def idx_slice(c):
            return pidx_v.at[pl.ds(pl.multiple_of(c * 128, 8), 128)]

        def start_gather(c, buf, sem):
            pltpu.async_copy(pairs_hbm.at[idx_slice(c)], buf, sem)

        def wait_gather(c, buf, sem):
            pltpu.make_async_copy(pairs_hbm.at[idx_slice(c)], buf, sem).wait()

        def out_slab(c):
            k_ = c // nbt
            bt = bbase // 128 + lax.rem(c, nbt)
            return out_hbm.at[k_, pl.ds(0, 8), pl.ds(bt, 1)]

        def start_out(c, tr, sem):
            pltpu.async_copy(tr, out_slab(c), sem)

        def wait_out(c, tr, sem):
            pltpu.make_async_copy(tr, out_slab(c), sem).wait()

        row_ids = [lax.iota(jnp.int32, _L) + g * _L for g in range(8)]

        def transpose(c, buf, tr):
            for g in range(8):
                poff = poff_v[pl.ds(c * 128 + g * _L, _L)]
                for d in range(_D):
                    vals = plsc.load_gather(buf, [row_ids[g], poff + d])
                    tr[d // 8, 0, d % 8, pl.ds(g * _L, _L)] = vals

        start_gather(0, rows0, gsem0)

        @pl.loop(0, nchunk, step=2)
        def pair_chunks(c):
            start_gather(c + 1, rows1, gsem1)
            wait_gather(c, rows0, gsem0)

            @pl.when(c >= 2)
            def _():
                wait_out(c - 2, tr0, osem0)

            transpose(c, rows0, tr0)
            start_out(c, tr0, osem0)

            @pl.when(c + 2 < nchunk)
            def _():
                start_gather(c + 2, rows0, gsem0)

            wait_gather(c + 1, rows1, gsem1)

            @pl.when(c >= 2)
            def _():
                wait_out(c - 1, tr1, osem1)

            transpose(c + 1, rows1, tr1)
            start_out(c + 1, tr1, osem1)

        wait_out(nchunk - 2, tr0, osem0)
        wait_out(nchunk - 1, tr1, osem1)

    return body(pairs, idsf)


def kernel(feats, ids):
    b, k = ids.shape
    idsf = ids.T.reshape(-1).astype(jnp.int32)
    pairs = feats.reshape(feats.shape[0] // 2, 128)
    out5 = _sc_gather(pairs, idsf, b, k)
    return out5.transpose(2, 4, 0, 1, 3).reshape(b, k, _D)


# speedup vs baseline: 1.2856x; 1.2856x over previous
import functools

import jax
import jax.numpy as jnp
from jax import lax
from jax.experimental import pallas as pl
from jax.experimental.pallas import tpu as pltpu
from jax.experimental.pallas import tpu_sc as plsc

_D = 64
_NC = 2
_NS = 16
_NW = _NC * _NS
_L = 16


def _sc_gather(pairs, idsf, b, k):
    n = b * k
    bb = b // _NW
    nbt = bb // 128
    nchunk = k * nbt
    bpw = n // _NW
    mesh = plsc.VectorSubcoreMesh(core_axis_name="c", subcore_axis_name="s")

    @functools.partial(
        pl.kernel,
        out_type=jax.ShapeDtypeStruct((k, _D // 8, b // 128, 8, 128), jnp.float32),
        mesh=mesh,
        scratch_types=[
            pltpu.VMEM((bpw,), jnp.int32),
            pltpu.VMEM((bpw,), jnp.int32),
            pltpu.VMEM((128, 128), jnp.float32),
            pltpu.VMEM((128, 128), jnp.float32),
            pltpu.VMEM((8, 1, 8, 128), jnp.float32),
            pltpu.VMEM((8, 1, 8, 128), jnp.float32),
            pltpu.SemaphoreType.DMA,
            pltpu.SemaphoreType.DMA,
            pltpu.SemaphoreType.DMA,
            pltpu.SemaphoreType.DMA,
        ],
        compiler_params=pltpu.CompilerParams(
            use_tc_tiling_on_sc=False,
            needs_layout_passes=False,
            disable_bounds_checks=True,
        ),
    )
    def body(pairs_hbm, ids_hbm, out_hbm, pidx_v, poff_v, rows0, rows1,
             tr0, tr1, gsem0, gsem1, osem0, osem1):
        wid = lax.axis_index("s") * _NC + lax.axis_index("c")
        bbase = wid * bb

        for k_ in range(k):
            pltpu.sync_copy(
                ids_hbm.at[pl.ds(k_ * b + bbase, bb)],
                pidx_v.at[pl.ds(k_ * bb, bb)],
            )

        @pl.loop(0, bpw // _L)
        def split(i):
            v = pidx_v[pl.ds(i * _L, _L)]
            poff_v[pl.ds(i * _L, _L)] = (v & 1) * _D
            pidx_v[pl.ds(i * _L, _L)] = lax.shift_right_logical(v, 1)

        def idx_slice(c):
            return pidx_v.at[pl.ds(pl.multiple_of(c * 128, 8), 128)]

        def start_gather(c, buf, sem):
            pltpu.async_copy(pairs_hbm.at[idx_slice(c)], buf, sem)

        def wait_gather(c, buf, sem):
            pltpu.make_async_copy(pairs_hbm.at[idx_slice(c)], buf, sem).wait()

        def out_slab(c):
            k_ = c // nbt
            bt = bbase // 128 + lax.rem(c, nbt)
            return out_hbm.at[k_, pl.ds(0, 8), pl.ds(bt, 1)]

        def start_out(c, tr, sem):
            pltpu.async_copy(tr, out_slab(c), sem)

        def wait_out(c, tr, sem):
            pltpu.make_async_copy(tr, out_slab(c), sem).wait()

        row_ids = [lax.iota(jnp.int32, _L) + g * _L for g in range(8)]

        def transpose(c, buf, tr):
            for g in range(8):
                poff = poff_v[pl.ds(c * 128 + g * _L, _L)]
                for d0 in range(0, _D, 8):
                    vals = [
                        plsc.load_gather(buf, [row_ids[g], poff + (d0 + i)])
                        for i in range(8)
                    ]
                    for i in range(8):
                        d = d0 + i
                        tr[d // 8, 0, d % 8, pl.ds(g * _L, _L)] = vals[i]

        start_gather(0, rows0, gsem0)

        @pl.loop(0, nchunk, step=2)
        def pair_chunks(c):
            start_gather(c + 1, rows1, gsem1)
            wait_gather(c, rows0, gsem0)

            @pl.when(c >= 2)
            def _():
                wait_out(c - 2, tr0, osem0)

            transpose(c, rows0, tr0)
            start_out(c, tr0, osem0)

            @pl.when(c + 2 < nchunk)
            def _():
                start_gather(c + 2, rows0, gsem0)

            wait_gather(c + 1, rows1, gsem1)

            @pl.when(c >= 2)
            def _():
                wait_out(c - 1, tr1, osem1)

            transpose(c + 1, rows1, tr1)
            start_out(c + 1, tr1, osem1)

        wait_out(nchunk - 2, tr0, osem0)
        wait_out(nchunk - 1, tr1, osem1)

    return body(pairs, idsf)


def kernel(feats, ids):
    b, k = ids.shape
    idsf = ids.T.reshape(-1).astype(jnp.int32)
    pairs = feats.reshape(feats.shape[0] // 2, 128)
    out5 = _sc_gather(pairs, idsf, b, k)
    return out5.transpose(2, 4, 0, 1, 3).reshape(b, k, _D)


# speedup vs baseline: 1.3207x; 1.0273x over previous
import functools

import jax
import jax.numpy as jnp
from jax import lax
from jax.experimental import pallas as pl
from jax.experimental.pallas import tpu as pltpu
from jax.experimental.pallas import tpu_sc as plsc

_D = 64
_NC = 2
_NS = 16
_NW = _NC * _NS
_L = 16


def _sc_gather(pairs, idsf, b, k):
    n = b * k
    bb = b // _NW
    nbt = bb // 128
    nchunk = k * nbt
    bpw = n // _NW
    mesh = plsc.VectorSubcoreMesh(core_axis_name="c", subcore_axis_name="s")

    @functools.partial(
        pl.kernel,
        out_type=jax.ShapeDtypeStruct((k, _D // 8, b // 128, 8, 128), jnp.float32),
        mesh=mesh,
        scratch_types=[
            pltpu.VMEM((bpw,), jnp.int32),
            pltpu.VMEM((bpw,), jnp.int32),
            pltpu.VMEM((128, 128), jnp.float32),
            pltpu.VMEM((128, 128), jnp.float32),
            pltpu.VMEM((8, 1, 8, 128), jnp.float32),
            pltpu.VMEM((8, 1, 8, 128), jnp.float32),
            pltpu.SemaphoreType.DMA,
            pltpu.SemaphoreType.DMA,
            pltpu.SemaphoreType.DMA,
            pltpu.SemaphoreType.DMA,
        ],
        compiler_params=pltpu.CompilerParams(
            use_tc_tiling_on_sc=False,
            needs_layout_passes=False,
            disable_bounds_checks=True,
        ),
    )
    def body(pairs_hbm, ids_hbm, out_hbm, pidx_v, poff_v, rows0, rows1,
             tr0, tr1, gsem0, gsem1, osem0, osem1):
        wid = lax.axis_index("s") * _NC + lax.axis_index("c")
        bbase = wid * bb

        for k_ in range(k):
            pltpu.async_copy(
                ids_hbm.at[pl.ds(k_ * b + bbase, bb)],
                pidx_v.at[pl.ds(k_ * bb, bb)],
                gsem0,
            )
        for k_ in range(k):
            pltpu.make_async_copy(
                ids_hbm.at[pl.ds(k_ * b + bbase, bb)],
                pidx_v.at[pl.ds(k_ * bb, bb)],
                gsem0,
            ).wait()

        @pl.loop(0, bpw // _L, unroll=4)
        def split(i):
            v = pidx_v[pl.ds(i * _L, _L)]
            poff_v[pl.ds(i * _L, _L)] = (v & 1) * _D
            pidx_v[pl.ds(i * _L, _L)] = lax.shift_right_logical(v, 1)

        def idx_slice(c):
            return pidx_v.at[pl.ds(pl.multiple_of(c * 128, 8), 128)]

        def start_gather(c, buf, sem):
            pltpu.async_copy(pairs_hbm.at[idx_slice(c)], buf, sem)

        def wait_gather(c, buf, sem):
            pltpu.make_async_copy(pairs_hbm.at[idx_slice(c)], buf, sem).wait()

        def out_slab(c):
            k_ = c // nbt
            bt = bbase // 128 + lax.rem(c, nbt)
            return out_hbm.at[k_, pl.ds(0, 8), pl.ds(bt, 1)]

        def start_out(c, tr, sem):
            pltpu.async_copy(tr, out_slab(c), sem)

        def wait_out(c, tr, sem):
            pltpu.make_async_copy(tr, out_slab(c), sem).wait()

        row_ids = [lax.iota(jnp.int32, _L) + g * _L for g in range(8)]

        def transpose(c, buf, tr):
            for g in range(8):
                poff = poff_v[pl.ds(c * 128 + g * _L, _L)]
                for d0 in range(0, _D, 16):
                    vals = [
                        plsc.load_gather(buf, [row_ids[g], poff + (d0 + i)])
                        for i in range(16)
                    ]
                    for i in range(16):
                        d = d0 + i
                        tr[d // 8, 0, d % 8, pl.ds(g * _L, _L)] = vals[i]

        start_gather(0, rows0, gsem0)

        @pl.loop(0, nchunk, step=2)
        def pair_chunks(c):
            start_gather(c + 1, rows1, gsem1)
            wait_gather(c, rows0, gsem0)

            @pl.when(c >= 2)
            def _():
                wait_out(c - 2, tr0, osem0)

            transpose(c, rows0, tr0)
            start_out(c, tr0, osem0)

            @pl.when(c + 2 < nchunk)
            def _():
                start_gather(c + 2, rows0, gsem0)

            wait_gather(c + 1, rows1, gsem1)

            @pl.when(c >= 2)
            def _():
                wait_out(c - 1, tr1, osem1)

            transpose(c + 1, rows1, tr1)
            start_out(c + 1, tr1, osem1)

        wait_out(nchunk - 2, tr0, osem0)
        wait_out(nchunk - 1, tr1, osem1)

    return body(pairs, idsf)


def kernel(feats, ids):
    b, k = ids.shape
    idsf = ids.T.reshape(-1).astype(jnp.int32)
    pairs = feats.reshape(feats.shape[0] // 2, 128)
    out5 = _sc_gather(pairs, idsf, b, k)
    return out5.transpose(2, 4, 0, 1, 3).reshape(b, k, _D)


# speedup vs baseline: 1.4318x; 1.0842x over previous
import functools

import jax
import jax.numpy as jnp
from jax import lax
from jax.experimental import pallas as pl
from jax.experimental.pallas import tpu as pltpu
from jax.experimental.pallas import tpu_sc as plsc

_D = 64
_NC = 2
_NS = 16
_NW = _NC * _NS


def _sc_gather(table, idsf, b, k):
    n = b * k
    bb = b // _NW
    bpw = n // _NW
    mesh = plsc.VectorSubcoreMesh(core_axis_name="c", subcore_axis_name="s")

    @functools.partial(
        pl.kernel,
        out_type=jax.ShapeDtypeStruct((k, b, _D), jnp.float32),
        mesh=mesh,
        scratch_types=[
            pltpu.VMEM((bpw,), jnp.int32),
            pltpu.VMEM((bb, _D), jnp.float32),
            pltpu.VMEM((bb, _D), jnp.float32),
            pltpu.SemaphoreType.DMA,
            pltpu.SemaphoreType.DMA,
        ],
        compiler_params=pltpu.CompilerParams(
            use_tc_tiling_on_sc=False,
            needs_layout_passes=False,
            disable_bounds_checks=True,
        ),
    )
    def body(table_hbm, ids_hbm, out_hbm, idx_v, rows0, rows1, gsem0, gsem1):
        wid = lax.axis_index("s") * _NC + lax.axis_index("c")
        bbase = pl.multiple_of(wid * bb, 8)

        for k_ in range(k):
            pltpu.async_copy(
                ids_hbm.at[pl.ds(k_ * b + bbase, bb)],
                idx_v.at[pl.ds(k_ * bb, bb)],
                gsem0,
            )
        for k_ in range(k):
            pltpu.make_async_copy(
                ids_hbm.at[pl.ds(k_ * b + bbase, bb)],
                idx_v.at[pl.ds(k_ * bb, bb)],
                gsem0,
            ).wait()

        def idx_slice(c):
            return idx_v.at[pl.ds(pl.multiple_of(c * bb, 8), bb)]

        def start_gather(c, buf, sem):
            pltpu.async_copy(table_hbm.at[idx_slice(c)], buf, sem)

        def wait_gather(c, buf, sem):
            pltpu.make_async_copy(table_hbm.at[idx_slice(c)], buf, sem).wait()

        def copy_out(c, buf):
            pltpu.sync_copy(buf, out_hbm.at[c, pl.ds(bbase, bb)])

        start_gather(0, rows0, gsem0)
        start_gather(1, rows1, gsem1)
        wait_gather(0, rows0, gsem0)
        copy_out(0, rows0)

        @pl.loop(1, k - 1, step=2)
        def pair(c):
            start_gather(c + 1, rows0, gsem0)
            wait_gather(c, rows1, gsem1)
            copy_out(c, rows1)
            start_gather(c + 2, rows1, gsem1)
            wait_gather(c + 1, rows0, gsem0)
            copy_out(c + 1, rows0)

        wait_gather(k - 1, rows1, gsem1)
        copy_out(k - 1, rows1)

    return body(table, idsf)


def kernel(feats, ids):
    b, k = ids.shape
    idsf = ids.T.reshape(-1).astype(jnp.int32)
    out3 = _sc_gather(feats, idsf, b, k)
    return out3.transpose(1, 0, 2)
